# baseline (speedup 1.0000x reference)
import jax
import jax.numpy as jnp
from jax.experimental import pallas as pl

_R, _C = 128, 100000
_BR = 8

_KS0 = 0
_KS1 = 1
_KS2 = _KS0 ^ _KS1 ^ 0x1BD11BDA


def _rotl(x, d):
    return jax.lax.shift_left(x, jnp.uint32(d)) | jax.lax.shift_right_logical(
        x, jnp.uint32(32 - d)
    )


def _threefry_xor_bits(cnt):
    ks = (jnp.uint32(_KS0), jnp.uint32(_KS1), jnp.uint32(_KS2))
    rots = ((13, 15, 26, 6), (17, 29, 16, 24))
    x0 = jnp.zeros_like(cnt) + ks[0]
    x1 = cnt + ks[1]
    for i in range(5):
        for d in rots[i % 2]:
            x0 = x0 + x1
            x1 = _rotl(x1, d)
            x1 = x0 ^ x1
        x0 = x0 + ks[(i + 1) % 3]
        x1 = x1 + ks[(i + 2) % 3] + jnp.uint32(i + 1)
    return x0 ^ x1


def _body(x_ref, o_ref):
    i = pl.program_id(0)
    row = jax.lax.broadcasted_iota(jnp.int32, (_BR, _C), 0) + i * _BR
    col = jax.lax.broadcasted_iota(jnp.int32, (_BR, _C), 1)
    cnt = (row * _C + col).astype(jnp.uint32)

    bits = _threefry_xor_bits(cnt)
    fb = jax.lax.shift_right_logical(bits, jnp.uint32(9)) | jnp.uint32(0x3F800000)
    u = jax.lax.bitcast_convert_type(fb, jnp.float32) - jnp.float32(1.0)

    eps = jnp.float32(1e-10)
    g = -jnp.log(eps - jnp.log(u + eps))

    p = x_ref[...] + g
    m = jnp.max(p, axis=1, keepdims=True)
    e = jnp.exp(p - m)
    s = jnp.sum(e, axis=1, keepdims=True)
    o_ref[...] = e * (jnp.float32(1.0) / s)


def kernel(logits):
    return pl.pallas_call(
        _body,
        grid=(_R // _BR,),
        in_specs=[pl.BlockSpec((_BR, _C), lambda i: (i, 0))],
        out_specs=pl.BlockSpec((_BR, _C), lambda i: (i, 0)),
        out_shape=jax.ShapeDtypeStruct((_R, _C), jnp.float32),
    )(logits)
